# baseline (speedup 1.0000x reference)
import functools

import jax
import jax.numpy as jnp
from jax import lax
from jax.experimental import pallas as pl
from jax.experimental.pallas import tpu as pltpu
from jax.experimental.pallas import tpu_sc as plsc

N_NODES = 10000
D = 128
NPAD = 10240
ROWS_PER_TILE = NPAD // 16
CHUNK = 40
NCHUNK = 64
NBUF = 4
ZROWS = 8
CROWS = 80
EDGES_PER_CORE = 40000
EDGES_PAD = 16 * NCHUNK * CHUNK


def _sc_accumulate(srcs, tgts, x_pad):
    mesh = plsc.VectorSubcoreMesh(core_axis_name="c", subcore_axis_name="s")

    @functools.partial(
        pl.kernel,
        out_type=(
            jax.ShapeDtypeStruct((2, NPAD, D), jnp.float32),
            jax.ShapeDtypeStruct((2, CROWS, 128), jnp.float32),
        ),
        mesh=mesh,
        compiler_params=pltpu.CompilerParams(needs_layout_passes=False),
        scratch_types=(
            pltpu.VMEM((NCHUNK, CHUNK), jnp.int32),
            pltpu.VMEM((NCHUNK, CHUNK), jnp.int32),
            *([pltpu.VMEM((CHUNK, D), jnp.float32)] * NBUF),
            pltpu.VMEM((ZROWS, D), jnp.float32),
            pltpu.VMEM((CROWS, 128), jnp.float32),
            pltpu.VMEM((CROWS,), jnp.int32),
            pltpu.VMEM_SHARED((NPAD, D), jnp.float32),
            pltpu.VMEM_SHARED((CROWS, 128), jnp.float32),
            *([pltpu.SemaphoreType.DMA] * (2 * NBUF)),
        ),
    )
    def k(srcs_hbm, tgts_hbm, x_hbm, agg_out, cnt_out,
          src_loc, tgt_loc, *rest):
        rows = rest[:NBUF]
        zrow, cnt_loc, ident, agg_sh, cnt_sh = rest[NBUF:NBUF + 5]
        gsems = rest[NBUF + 5:2 * NBUF + 5]
        ssems = rest[2 * NBUF + 5:]
        c = lax.axis_index("c")
        s = lax.axis_index("s")
        row0 = s * ROWS_PER_TILE

        zero16 = jnp.zeros((16,), jnp.float32)
        iota16 = lax.iota(jnp.int32, 16)

        def fill_zrow(i, carry):
            for q in range(D // 16):
                zrow[i, pl.ds(q * 16, 16)] = zero16
            return carry
        lax.fori_loop(0, ZROWS, fill_zrow, 0)

        def fill_cnt(i, carry):
            for q in range(128 // 16):
                cnt_loc[i, pl.ds(q * 16, 16)] = zero16
            return carry
        lax.fori_loop(0, CROWS, fill_cnt, 0)

        for q in range(CROWS // 16):
            ident[pl.ds(q * 16, 16)] = iota16 + (q * 16)

        def zero_sh(q, carry):
            pltpu.sync_copy(zrow, agg_sh.at[pl.ds(row0 + q * ZROWS, ZROWS)])
            return carry
        lax.fori_loop(0, ROWS_PER_TILE // ZROWS, zero_sh, 0)

        @pl.when(s < CROWS // 8)
        def _():
            pltpu.sync_copy(zrow.at[pl.ds(0, 8)],
                            cnt_sh.at[pl.ds(s * 8, 8)])

        pltpu.sync_copy(srcs_hbm.at[c, s], src_loc)
        pltpu.sync_copy(tgts_hbm.at[c, s], tgt_loc)
        plsc.subcore_barrier()

        one16 = jnp.ones((16,), jnp.float32)

        def counts(j):
            for q in range(CHUNK // 16):
                idx16 = src_loc[j, pl.ds(q * 16, 16)]
                r = lax.shift_right_logical(idx16, 7)
                col = jnp.bitwise_and(idx16, 127)
                plsc.addupdate_scatter(cnt_loc, [r, col], one16)

        for k in range(NBUF):
            pltpu.async_copy(x_hbm.at[tgt_loc.at[k]], rows[k], gsems[k])

        def step(i, carry):
            j0 = NBUF * i
            scs = []
            for k in range(NBUF):
                j = j0 + k
                pltpu.make_async_copy(x_hbm.at[tgt_loc.at[j]], rows[k],
                                      gsems[k]).wait()
                scs.append(pltpu.async_copy(rows[k], agg_sh.at[src_loc.at[j]],
                                            ssems[k], add=True))
                counts(j)
            for k in range(NBUF):
                j = j0 + k
                scs[k].wait()

                @pl.when(j + NBUF < NCHUNK)
                def _(k=k, j=j):
                    pltpu.async_copy(x_hbm.at[tgt_loc.at[j + NBUF]], rows[k],
                                     gsems[k])
            return carry
        lax.fori_loop(0, NCHUNK // NBUF, step, 0)

        pltpu.sync_copy(cnt_loc, cnt_sh.at[ident], add=True)
        plsc.subcore_barrier()

        pltpu.sync_copy(agg_sh.at[pl.ds(row0, ROWS_PER_TILE)],
                        agg_out.at[c, pl.ds(row0, ROWS_PER_TILE)])
        @pl.when(s < CROWS // 8)
        def _():
            pltpu.sync_copy(cnt_sh.at[pl.ds(s * 8, 8)],
                            cnt_out.at[c, pl.ds(s * 8, 8)])

    return k(srcs, tgts, x_pad)


def _col(cnt2d):
    t = jnp.transpose(cnt2d)
    return jnp.concatenate([t[:, j:j + 1] for j in range(cnt2d.shape[0])],
                           axis=0)


def _tc_body(agg_ref, cnt_ref, w0t, w1t, wg0t, wg1t, b0r, b1r, bgr, o_ref):
    a4 = agg_ref[0]
    ad = agg_ref[1]
    c4 = _col(cnt_ref[0]) + 1e-6
    c8 = c4 + _col(cnt_ref[1])
    hp = jax.lax.Precision.DEFAULT
    A4 = a4 / c4
    A8 = (a4 + ad) / c8
    out0 = jnp.dot(A4, w0t[...], preferred_element_type=jnp.float32, precision=hp) + b0r[...]
    out1 = jnp.dot(A8, w1t[...], preferred_element_type=jnp.float32, precision=hp) + b1r[...]
    g = jax.nn.sigmoid(
        jnp.dot(out0, wg0t[...], preferred_element_type=jnp.float32, precision=hp)
        + jnp.dot(out1, wg1t[...], preferred_element_type=jnp.float32, precision=hp)
        + bgr[...])
    o_ref[...] = g * out0 + (1.0 - g) * out1


def _tc_fuse(agg, cnt, w0t, w1t, wg0t, wg1t, b0r, b1r, bgr):
    bm = 1024
    grid = (NPAD // bm,)
    full_w = pl.BlockSpec((D, D), lambda i: (0, 0))
    full_b = pl.BlockSpec((1, D), lambda i: (0, 0))
    return pl.pallas_call(
        _tc_body,
        grid=grid,
        in_specs=[
            pl.BlockSpec((2, bm, D), lambda i: (0, i, 0)),
            pl.BlockSpec((2, bm // 128, 128), lambda i: (0, i, 0)),
            full_w, full_w, full_w, full_w, full_b, full_b, full_b,
        ],
        out_specs=pl.BlockSpec((bm, D), lambda i: (i, 0)),
        out_shape=jax.ShapeDtypeStruct((N_NODES, D), jnp.float32),
    )(agg, cnt, w0t, w1t, wg0t, wg1t, b0r, b1r, bgr)


def kernel(x, edge_index, W0, b0, W1, b1, Wg, bg):
    src = edge_index[0, :2 * EDGES_PER_CORE].astype(jnp.int32)
    tgt = edge_index[1, :2 * EDGES_PER_CORE].astype(jnp.int32)

    pad_src = N_NODES + (jnp.arange(EDGES_PAD - EDGES_PER_CORE,
                                    dtype=jnp.int32) % (NPAD - N_NODES))
    pad_tgt = jnp.zeros((EDGES_PAD - EDGES_PER_CORE,), jnp.int32)
    srcs = jnp.stack([
        jnp.concatenate([src[:EDGES_PER_CORE], pad_src]),
        jnp.concatenate([src[EDGES_PER_CORE:], pad_src]),
    ]).reshape(2, 16, NCHUNK, CHUNK)
    tgts = jnp.stack([
        jnp.concatenate([tgt[:EDGES_PER_CORE], pad_tgt]),
        jnp.concatenate([tgt[EDGES_PER_CORE:], pad_tgt]),
    ]).reshape(2, 16, NCHUNK, CHUNK)

    agg, cnt = _sc_accumulate(srcs, tgts, x)

    return _tc_fuse(
        agg, cnt,
        W0.T, W1.T, Wg[:, :D].T, Wg[:, D:].T,
        b0[None, :], b1[None, :], bg[None, :],
    )


# speedup vs baseline: 1.0099x; 1.0099x over previous
import functools

import jax
import jax.numpy as jnp
from jax import lax
from jax.experimental import pallas as pl
from jax.experimental.pallas import tpu as pltpu
from jax.experimental.pallas import tpu_sc as plsc

N_NODES = 10000
D = 128
NPAD = 10240
ROWS_PER_TILE = NPAD // 16
CHUNK = 80
NCHUNK = 32
ZROWS = 32
CROWS = 80
EDGES_PER_CORE = 40000
EDGES_PAD = 16 * NCHUNK * CHUNK


def _sc_accumulate(srcs, tgts, x_pad):
    mesh = plsc.VectorSubcoreMesh(core_axis_name="c", subcore_axis_name="s")

    @functools.partial(
        pl.kernel,
        out_type=(
            jax.ShapeDtypeStruct((2, NPAD, D), jnp.float32),
            jax.ShapeDtypeStruct((2, CROWS, 128), jnp.float32),
        ),
        mesh=mesh,
        compiler_params=pltpu.CompilerParams(needs_layout_passes=False),
        scratch_types=(
            pltpu.VMEM((NCHUNK, CHUNK), jnp.int32),
            pltpu.VMEM((NCHUNK, CHUNK), jnp.int32),
            pltpu.VMEM((CHUNK, D), jnp.float32),
            pltpu.VMEM((CHUNK, D), jnp.float32),
            pltpu.VMEM((ZROWS, D), jnp.float32),
            pltpu.VMEM((CROWS, 128), jnp.float32),
            pltpu.VMEM((CROWS,), jnp.int32),
            pltpu.VMEM_SHARED((NPAD, D), jnp.float32),
            pltpu.VMEM_SHARED((CROWS, 128), jnp.float32),
            pltpu.SemaphoreType.DMA,
            pltpu.SemaphoreType.DMA,
            pltpu.SemaphoreType.DMA,
            pltpu.SemaphoreType.DMA,
        ),
    )
    def k(srcs_hbm, tgts_hbm, x_hbm, agg_out, cnt_out,
          src_loc, tgt_loc, rows_a, rows_b, zrow, cnt_loc, ident,
          agg_sh, cnt_sh, gsem_a, gsem_b, ssem_a, ssem_b):
        c = lax.axis_index("c")
        s = lax.axis_index("s")
        row0 = s * ROWS_PER_TILE

        zero16 = jnp.zeros((16,), jnp.float32)
        iota16 = lax.iota(jnp.int32, 16)

        def fill_zrow(i, carry):
            for q in range(D // 16):
                zrow[i, pl.ds(q * 16, 16)] = zero16
            return carry
        lax.fori_loop(0, ZROWS, fill_zrow, 0)

        def fill_cnt(i, carry):
            for q in range(128 // 16):
                cnt_loc[i, pl.ds(q * 16, 16)] = zero16
            return carry
        lax.fori_loop(0, CROWS, fill_cnt, 0)

        for q in range(CROWS // 16):
            ident[pl.ds(q * 16, 16)] = iota16 + (q * 16)

        pltpu.sync_copy(srcs_hbm.at[c, s], src_loc)
        pltpu.sync_copy(tgts_hbm.at[c, s], tgt_loc)
        pltpu.async_copy(x_hbm.at[tgt_loc.at[0]], rows_a, gsem_a)
        pltpu.async_copy(x_hbm.at[tgt_loc.at[1]], rows_b, gsem_b)

        def zero_sh(q, carry):
            pltpu.sync_copy(zrow, agg_sh.at[pl.ds(row0 + q * ZROWS, ZROWS)])
            return carry
        lax.fori_loop(0, ROWS_PER_TILE // ZROWS, zero_sh, 0)

        @pl.when(s < CROWS // 8)
        def _():
            pltpu.sync_copy(zrow.at[pl.ds(0, 8)],
                            cnt_sh.at[pl.ds(s * 8, 8)])
        plsc.subcore_barrier()

        one16 = jnp.ones((16,), jnp.float32)

        def counts(j):
            for q in range(CHUNK // 16):
                idx16 = src_loc[j, pl.ds(q * 16, 16)]
                r = lax.shift_right_logical(idx16, 7)
                col = jnp.bitwise_and(idx16, 127)
                plsc.addupdate_scatter(cnt_loc, [r, col], one16)

        def step(i, carry):
            j0 = 2 * i
            j1 = j0 + 1
            pltpu.make_async_copy(x_hbm.at[tgt_loc.at[j0]], rows_a, gsem_a).wait()
            sc_a = pltpu.async_copy(rows_a, agg_sh.at[src_loc.at[j0]],
                                    ssem_a, add=True)
            counts(j0)
            pltpu.make_async_copy(x_hbm.at[tgt_loc.at[j1]], rows_b, gsem_b).wait()
            sc_b = pltpu.async_copy(rows_b, agg_sh.at[src_loc.at[j1]],
                                    ssem_b, add=True)
            counts(j1)
            sc_a.wait()

            @pl.when(j0 + 2 < NCHUNK)
            def _():
                pltpu.async_copy(x_hbm.at[tgt_loc.at[j0 + 2]], rows_a, gsem_a)
            sc_b.wait()

            @pl.when(j1 + 2 < NCHUNK)
            def _():
                pltpu.async_copy(x_hbm.at[tgt_loc.at[j1 + 2]], rows_b, gsem_b)
            return carry
        lax.fori_loop(0, NCHUNK // 2, step, 0)

        pltpu.sync_copy(cnt_loc, cnt_sh.at[ident], add=True)
        plsc.subcore_barrier()

        pltpu.sync_copy(agg_sh.at[pl.ds(row0, ROWS_PER_TILE)],
                        agg_out.at[c, pl.ds(row0, ROWS_PER_TILE)])
        @pl.when(s < CROWS // 8)
        def _():
            pltpu.sync_copy(cnt_sh.at[pl.ds(s * 8, 8)],
                            cnt_out.at[c, pl.ds(s * 8, 8)])

    return k(srcs, tgts, x_pad)


def _col(cnt2d):
    t = jnp.transpose(cnt2d)
    return jnp.concatenate([t[:, j:j + 1] for j in range(cnt2d.shape[0])],
                           axis=0)


def _tc_body(agg_ref, cnt_ref, w0t, w1t, wg0t, wg1t, b0r, b1r, bgr, o_ref):
    a4 = agg_ref[0]
    ad = agg_ref[1]
    c4 = _col(cnt_ref[0]) + 1e-6
    c8 = c4 + _col(cnt_ref[1])
    hp = jax.lax.Precision.DEFAULT
    A4 = a4 / c4
    A8 = (a4 + ad) / c8
    out0 = jnp.dot(A4, w0t[...], preferred_element_type=jnp.float32, precision=hp) + b0r[...]
    out1 = jnp.dot(A8, w1t[...], preferred_element_type=jnp.float32, precision=hp) + b1r[...]
    g = jax.nn.sigmoid(
        jnp.dot(out0, wg0t[...], preferred_element_type=jnp.float32, precision=hp)
        + jnp.dot(out1, wg1t[...], preferred_element_type=jnp.float32, precision=hp)
        + bgr[...])
    o_ref[...] = g * out0 + (1.0 - g) * out1


def _tc_fuse(agg, cnt, w0t, w1t, wg0t, wg1t, b0r, b1r, bgr):
    bm = 1024
    grid = (NPAD // bm,)
    full_w = pl.BlockSpec((D, D), lambda i: (0, 0))
    full_b = pl.BlockSpec((1, D), lambda i: (0, 0))
    return pl.pallas_call(
        _tc_body,
        grid=grid,
        in_specs=[
            pl.BlockSpec((2, bm, D), lambda i: (0, i, 0)),
            pl.BlockSpec((2, bm // 128, 128), lambda i: (0, i, 0)),
            full_w, full_w, full_w, full_w, full_b, full_b, full_b,
        ],
        out_specs=pl.BlockSpec((bm, D), lambda i: (i, 0)),
        out_shape=jax.ShapeDtypeStruct((N_NODES, D), jnp.float32),
    )(agg, cnt, w0t, w1t, wg0t, wg1t, b0r, b1r, bgr)


def kernel(x, edge_index, W0, b0, W1, b1, Wg, bg):
    src = edge_index[0, :2 * EDGES_PER_CORE].astype(jnp.int32)
    tgt = edge_index[1, :2 * EDGES_PER_CORE].astype(jnp.int32)

    pad_src = N_NODES + (jnp.arange(EDGES_PAD - EDGES_PER_CORE,
                                    dtype=jnp.int32) % (NPAD - N_NODES))
    pad_tgt = jnp.zeros((EDGES_PAD - EDGES_PER_CORE,), jnp.int32)
    srcs = jnp.stack([
        jnp.concatenate([src[:EDGES_PER_CORE], pad_src]),
        jnp.concatenate([src[EDGES_PER_CORE:], pad_src]),
    ]).reshape(2, 16, NCHUNK, CHUNK)
    tgts = jnp.stack([
        jnp.concatenate([tgt[:EDGES_PER_CORE], pad_tgt]),
        jnp.concatenate([tgt[EDGES_PER_CORE:], pad_tgt]),
    ]).reshape(2, 16, NCHUNK, CHUNK)

    agg, cnt = _sc_accumulate(srcs, tgts, x)

    return _tc_fuse(
        agg, cnt,
        W0.T, W1.T, Wg[:, :D].T, Wg[:, D:].T,
        b0[None, :], b1[None, :], bg[None, :],
    )


# speedup vs baseline: 1.0402x; 1.0300x over previous
import functools

import jax
import jax.numpy as jnp
from jax import lax
from jax.experimental import pallas as pl
from jax.experimental.pallas import tpu as pltpu
from jax.experimental.pallas import tpu_sc as plsc

N_NODES = 10000
D = 128
NPAD = 10240
ROWS_PER_TILE = NPAD // 16
CHUNK = 128
NCHUNK = 20
CROWS = 80
EDGES_PER_CORE = 40000
EDGES_PAD = 16 * NCHUNK * CHUNK


def _sc_accumulate(edges, x):
    mesh = plsc.VectorSubcoreMesh(core_axis_name="c", subcore_axis_name="s")

    @functools.partial(
        pl.kernel,
        out_type=(
            jax.ShapeDtypeStruct((2, NPAD, D), jnp.float32),
            jax.ShapeDtypeStruct((2, CROWS, 128), jnp.float32),
        ),
        mesh=mesh,
        compiler_params=pltpu.CompilerParams(needs_layout_passes=False),
        scratch_types=(
            pltpu.VMEM((2 * NCHUNK, CHUNK), jnp.int32),
            pltpu.VMEM((CHUNK, D), jnp.float32),
            pltpu.VMEM((CHUNK, D), jnp.float32),
            pltpu.VMEM((CROWS, 128), jnp.float32),
            pltpu.VMEM_SHARED((NPAD, D), jnp.float32),
            pltpu.VMEM_SHARED((CROWS, 128), jnp.float32),
            pltpu.SemaphoreType.DMA,
            pltpu.SemaphoreType.DMA,
            pltpu.SemaphoreType.DMA,
            pltpu.SemaphoreType.DMA,
        ),
    )
    def k(edges_hbm, x_hbm, agg_out, cnt_out,
          idx_loc, rows_a, rows_b, cnt_loc,
          agg_sh, cnt_sh, gsem_a, gsem_b, ssem_a, ssem_b):
        c = lax.axis_index("c")
        s = lax.axis_index("s")
        row0 = s * ROWS_PER_TILE

        zero16 = jnp.zeros((16,), jnp.float32)
        one16 = jnp.ones((16,), jnp.float32)
        iota16 = lax.iota(jnp.int32, 16)

        pltpu.sync_copy(edges_hbm.at[c, s], idx_loc)
        pltpu.async_copy(x_hbm.at[idx_loc.at[3]], rows_b, gsem_b)

        def fill_zero(i, carry):
            for q in range(D // 16):
                rows_a[i, pl.ds(q * 16, 16)] = zero16
            return carry
        lax.fori_loop(0, CHUNK, fill_zero, 0)

        def zero_sh(q, carry):
            pltpu.sync_copy(rows_a, agg_sh.at[pl.ds(row0 + q * CHUNK, CHUNK)])
            return carry
        lax.fori_loop(0, ROWS_PER_TILE // CHUNK, zero_sh, 0)

        @pl.when(s < CROWS // 8)
        def _():
            pltpu.sync_copy(rows_a.at[pl.ds(0, 8)],
                            cnt_sh.at[pl.ds(s * 8, 8)])

        pltpu.async_copy(x_hbm.at[idx_loc.at[1]], rows_a, gsem_a)

        def fill_cnt(i, carry):
            for q in range(128 // 16):
                cnt_loc[i, pl.ds(q * 16, 16)] = zero16
            return carry
        lax.fori_loop(0, CROWS, fill_cnt, 0)
        plsc.subcore_barrier()

        def counts(jr):
            for q in range(CHUNK // 16):
                idx16 = idx_loc[jr, pl.ds(q * 16, 16)]
                r = lax.shift_right_logical(idx16, 7)
                col = jnp.bitwise_and(idx16, 127)
                plsc.addupdate_scatter(cnt_loc, [r, col], one16)

        def step(i, carry):
            r0 = 4 * i
            pltpu.make_async_copy(x_hbm.at[idx_loc.at[r0 + 1]], rows_a,
                                  gsem_a).wait()
            sc_a = pltpu.async_copy(rows_a, agg_sh.at[idx_loc.at[r0]],
                                    ssem_a, add=True)
            counts(r0)
            pltpu.make_async_copy(x_hbm.at[idx_loc.at[r0 + 3]], rows_b,
                                  gsem_b).wait()
            sc_b = pltpu.async_copy(rows_b, agg_sh.at[idx_loc.at[r0 + 2]],
                                    ssem_b, add=True)
            counts(r0 + 2)
            sc_a.wait()

            @pl.when(r0 + 5 < 2 * NCHUNK)
            def _():
                pltpu.async_copy(x_hbm.at[idx_loc.at[r0 + 5]], rows_a, gsem_a)
            sc_b.wait()

            @pl.when(r0 + 7 < 2 * NCHUNK)
            def _():
                pltpu.async_copy(x_hbm.at[idx_loc.at[r0 + 7]], rows_b, gsem_b)
            return carry
        lax.fori_loop(0, NCHUNK // 2, step, 0)

        for q in range(CROWS // 16):
            pltpu.sync_copy(cnt_loc.at[pl.ds(q * 16, 16)],
                            cnt_sh.at[iota16 + q * 16], add=True)
        plsc.subcore_barrier()

        pltpu.sync_copy(agg_sh.at[pl.ds(row0, ROWS_PER_TILE)],
                        agg_out.at[c, pl.ds(row0, ROWS_PER_TILE)])

        @pl.when(s < CROWS // 8)
        def _():
            pltpu.sync_copy(cnt_sh.at[pl.ds(s * 8, 8)],
                            cnt_out.at[c, pl.ds(s * 8, 8)])

    return k(edges, x)


def _col(cnt2d):
    t = jnp.transpose(cnt2d)
    return jnp.concatenate([t[:, j:j + 1] for j in range(cnt2d.shape[0])],
                           axis=0)


def _tc_body(agg_ref, cnt_ref, w0t, w1t, wg0t, wg1t, b0r, b1r, bgr, o_ref):
    a4 = agg_ref[0]
    ad = agg_ref[1]
    c4 = _col(cnt_ref[0]) + 1e-6
    c8 = c4 + _col(cnt_ref[1])
    hp = jax.lax.Precision.DEFAULT
    A4 = a4 / c4
    A8 = (a4 + ad) / c8
    out0 = jnp.dot(A4, w0t[...], preferred_element_type=jnp.float32, precision=hp) + b0r[...]
    out1 = jnp.dot(A8, w1t[...], preferred_element_type=jnp.float32, precision=hp) + b1r[...]
    g = jax.nn.sigmoid(
        jnp.dot(out0, wg0t[...], preferred_element_type=jnp.float32, precision=hp)
        + jnp.dot(out1, wg1t[...], preferred_element_type=jnp.float32, precision=hp)
        + bgr[...])
    o_ref[...] = g * out0 + (1.0 - g) * out1


def _tc_fuse(agg, cnt, w0t, w1t, wg0t, wg1t, b0r, b1r, bgr):
    bm = 1024
    grid = (NPAD // bm,)
    full_w = pl.BlockSpec((D, D), lambda i: (0, 0))
    full_b = pl.BlockSpec((1, D), lambda i: (0, 0))
    return pl.pallas_call(
        _tc_body,
        grid=grid,
        in_specs=[
            pl.BlockSpec((2, bm, D), lambda i: (0, i, 0)),
            pl.BlockSpec((2, bm // 128, 128), lambda i: (0, i, 0)),
            full_w, full_w, full_w, full_w, full_b, full_b, full_b,
        ],
        out_specs=pl.BlockSpec((bm, D), lambda i: (i, 0)),
        out_shape=jax.ShapeDtypeStruct((N_NODES, D), jnp.float32),
    )(agg, cnt, w0t, w1t, wg0t, wg1t, b0r, b1r, bgr)


def kernel(x, edge_index, W0, b0, W1, b1, Wg, bg):
    src = edge_index[0, :2 * EDGES_PER_CORE].astype(jnp.int32)
    tgt = edge_index[1, :2 * EDGES_PER_CORE].astype(jnp.int32)

    pad_src = N_NODES + (jnp.arange(EDGES_PAD - EDGES_PER_CORE,
                                    dtype=jnp.int32) % (NPAD - N_NODES))
    pad_tgt = jnp.zeros((EDGES_PAD - EDGES_PER_CORE,), jnp.int32)
    srcs = jnp.stack([
        jnp.concatenate([src[:EDGES_PER_CORE], pad_src]),
        jnp.concatenate([src[EDGES_PER_CORE:], pad_src]),
    ]).reshape(2, 16, NCHUNK, CHUNK)
    tgts = jnp.stack([
        jnp.concatenate([tgt[:EDGES_PER_CORE], pad_tgt]),
        jnp.concatenate([tgt[EDGES_PER_CORE:], pad_tgt]),
    ]).reshape(2, 16, NCHUNK, CHUNK)
    edges = jnp.stack([srcs, tgts], axis=3).reshape(2, 16, 2 * NCHUNK, CHUNK)

    agg, cnt = _sc_accumulate(edges, x)

    return _tc_fuse(
        agg, cnt,
        W0.T, W1.T, Wg[:, :D].T, Wg[:, D:].T,
        b0[None, :], b1[None, :], bg[None, :],
    )
